# baseline (speedup 1.0000x reference)
import jax
import jax.numpy as jnp
from jax.experimental import pallas as pl
from jax.experimental.pallas import tpu as pltpu

E = 64
TOPK = 2
D = 1024
FF = 768
CAP = 16
ALPHA = 1.702
LIMIT = 7.0
T = 128
FH = FF // 2


def _moe_kernel(hid_ref, logit_ref, gate_ref, up_ref, dp_ref, out_ref,
                s_ref, xe_ref):
    step = pl.program_id(0)
    se = jnp.maximum(step, 1) - 1
    e = se // 2
    h = se % 2

    @pl.when(step == 0)
    def _routing():
        logits = logit_ref[...]
        col = jax.lax.broadcasted_iota(jnp.int32, (T, E), 1)
        w1 = jnp.max(logits, axis=-1, keepdims=True)
        i1 = jnp.min(jnp.where(logits >= w1, col, E), axis=-1, keepdims=True)
        logits2 = jnp.where(col == i1, -1e30, logits)
        w2 = jnp.max(logits2, axis=-1, keepdims=True)
        i2 = jnp.min(jnp.where(logits2 >= w2, col, E), axis=-1, keepdims=True)
        onehot1 = (col == i1).astype(jnp.float32)
        onehot2 = (col == i2).astype(jnp.float32)
        cnt = onehot1 + onehot2
        ltri = (jax.lax.broadcasted_iota(jnp.int32, (T, T), 0)
                > jax.lax.broadcasted_iota(jnp.int32, (T, T), 1)
                ).astype(jnp.float32)
        cex = jnp.dot(ltri, cnt, preferred_element_type=jnp.float32)
        pos1 = jnp.sum(cex * onehot1, axis=-1, keepdims=True)
        pos2 = jnp.sum(cex * onehot2, axis=-1, keepdims=True)
        pos1 = jnp.where(pos1 < CAP, pos1, 255.0)
        pos2 = jnp.where(pos2 < CAP, pos2, 255.0)
        w1n = jax.nn.sigmoid(w1 - w2)
        w2n = 1.0 - w1n
        s_ref[...] = jnp.concatenate(
            [i1.astype(jnp.float32), i2.astype(jnp.float32),
             pos1, pos2, w1n, w2n, w1n, w2n], axis=1)

    @pl.when(step > 0)
    def _expert():
        _expert_step(step, e, h, hid_ref, gate_ref, up_ref, dp_ref, out_ref,
                     s_ref, xe_ref)


def _expert_step(step, e, h, hid_ref, gate_ref, up_ref, dp_ref, out_ref,
                 s_ref, xe_ref):
    ef = e.astype(jnp.float32)
    i1f = s_ref[:, 0:1]
    i2f = s_ref[:, 1:2]
    pos1f = s_ref[:, 2:3]
    pos2f = s_ref[:, 3:4]
    w1f = s_ref[:, 4:5]
    w2f = s_ref[:, 5:6]
    capcol = jax.lax.broadcasted_iota(jnp.int32, (T, CAP), 1).astype(jnp.float32)
    sel1 = ((i1f == ef) & (pos1f == capcol)).astype(jnp.float32)
    sel2 = ((i2f == ef) & (pos2f == capcol)).astype(jnp.float32)
    selw = sel1 * w1f + sel2 * w2f

    @pl.when(h == 0)
    def _dispatch():
        sel = sel1 + sel2
        xe_ref[...] = jax.lax.dot_general(
            sel, hid_ref[...], (((0,), (0,)), ((), ())),
            preferred_element_type=jnp.float32).astype(jnp.bfloat16)

    xe = xe_ref[...]
    gu = jnp.dot(xe, gate_ref[0].astype(jnp.bfloat16),
                 preferred_element_type=jnp.float32)
    upv = jnp.dot(xe, up_ref[0].astype(jnp.bfloat16),
                  preferred_element_type=jnp.float32)
    gate = jnp.minimum(gu, LIMIT)
    up = jnp.clip(upv, -LIMIT, LIMIT)
    glu = gate * jax.nn.sigmoid(gate * ALPHA)
    act = (up + 1.0) * glu
    out_b = jnp.dot(act.astype(jnp.bfloat16), dp_ref[0].astype(jnp.bfloat16),
                    preferred_element_type=jnp.float32)
    contrib = jnp.dot(selw, out_b, preferred_element_type=jnp.float32)

    @pl.when(step == 1)
    def _init():
        out_ref[...] = contrib

    @pl.when(step > 1)
    def _acc():
        out_ref[...] += contrib


def kernel(hidden_states, router_logits, gate_up_proj, down_proj):
    def emap(s):
        se = jnp.maximum(s, 1) - 1
        return se // 2, se % 2

    def gatemap(s):
        e, h = emap(s)
        return e, 0, h

    def upmap(s):
        e, h = emap(s)
        return e, 0, 2 + h

    def dpmap(s):
        e, h = emap(s)
        return e, h, 0

    return pl.pallas_call(
        _moe_kernel,
        grid=(2 * E + 1,),
        in_specs=[
            pl.BlockSpec((T, D), lambda s: (0, 0)),
            pl.BlockSpec((T, E), lambda s: (0, 0)),
            pl.BlockSpec((1, D, FH), gatemap),
            pl.BlockSpec((1, D, FH), upmap),
            pl.BlockSpec((1, FH, D), dpmap),
        ],
        out_specs=pl.BlockSpec((T, D), lambda s: (0, 0)),
        out_shape=jax.ShapeDtypeStruct((T, D), jnp.float32),
        scratch_shapes=[
            pltpu.VMEM((T, 8), jnp.float32),
            pltpu.VMEM((CAP, D), jnp.bfloat16),
        ],
        compiler_params=pltpu.CompilerParams(
            dimension_semantics=("arbitrary",),
        ),
    )(hidden_states, router_logits, gate_up_proj, gate_up_proj, down_proj)


# speedup vs baseline: 1.3127x; 1.3127x over previous
import jax
import jax.numpy as jnp
from jax.experimental import pallas as pl
from jax.experimental.pallas import tpu as pltpu

E = 64
TOPK = 2
D = 1024
FF = 768
CAP = 16
ALPHA = 1.702
LIMIT = 7.0
T = 128


def _moe_kernel(hid_ref, logit_ref, gu_ref, dp_ref, out_ref, s_ref):
    step = pl.program_id(0)
    e = jnp.maximum(step, 1) - 1

    @pl.when(step == 0)
    def _routing():
        logits = logit_ref[...]
        col = jax.lax.broadcasted_iota(jnp.int32, (T, E), 1)
        w1 = jnp.max(logits, axis=-1, keepdims=True)
        i1 = jnp.min(jnp.where(logits >= w1, col, E), axis=-1, keepdims=True)
        logits2 = jnp.where(col == i1, -1e30, logits)
        w2 = jnp.max(logits2, axis=-1, keepdims=True)
        i2 = jnp.min(jnp.where(logits2 >= w2, col, E), axis=-1, keepdims=True)
        onehot1 = (col == i1).astype(jnp.float32)
        onehot2 = (col == i2).astype(jnp.float32)
        cnt = onehot1 + onehot2
        ltri = (jax.lax.broadcasted_iota(jnp.int32, (T, T), 0)
                > jax.lax.broadcasted_iota(jnp.int32, (T, T), 1)
                ).astype(jnp.float32)
        cex = jnp.dot(ltri, cnt, preferred_element_type=jnp.float32)
        pos1 = jnp.sum(cex * onehot1, axis=-1, keepdims=True)
        pos2 = jnp.sum(cex * onehot2, axis=-1, keepdims=True)
        pos1 = jnp.where(pos1 < CAP, pos1, 255.0)
        pos2 = jnp.where(pos2 < CAP, pos2, 255.0)
        w1n = jax.nn.sigmoid(w1 - w2)
        w2n = 1.0 - w1n
        s_ref[...] = jnp.concatenate(
            [i1.astype(jnp.float32), i2.astype(jnp.float32),
             pos1, pos2, w1n, w2n, w1n, w2n], axis=1)

    @pl.when(step > 0)
    def _expert():
        _expert_step(e, hid_ref, gu_ref, dp_ref, out_ref, s_ref)


def _expert_step(e, hid_ref, gu_ref, dp_ref, out_ref, s_ref):
    ef = e.astype(jnp.float32)
    i1f = s_ref[:, 0:1]
    i2f = s_ref[:, 1:2]
    pos1f = s_ref[:, 2:3]
    pos2f = s_ref[:, 3:4]
    w1f = s_ref[:, 4:5]
    w2f = s_ref[:, 5:6]
    capcol = jax.lax.broadcasted_iota(jnp.int32, (T, CAP), 1).astype(jnp.float32)
    sel1 = ((i1f == ef) & (pos1f == capcol)).astype(jnp.float32)
    sel2 = ((i2f == ef) & (pos2f == capcol)).astype(jnp.float32)
    sel = sel1 + sel2
    selw = sel1 * w1f + sel2 * w2f

    hid = hid_ref[...]
    xe = jax.lax.dot_general(sel, hid, (((0,), (0,)), ((), ())),
                             preferred_element_type=jnp.float32)
    gu = jnp.dot(xe.astype(jnp.bfloat16), gu_ref[0].astype(jnp.bfloat16),
                 preferred_element_type=jnp.float32)
    gate = jnp.minimum(gu[:, :FF], LIMIT)
    up = jnp.clip(gu[:, FF:], -LIMIT, LIMIT)
    glu = gate * jax.nn.sigmoid(gate * ALPHA)
    act = (up + 1.0) * glu
    out_b = jnp.dot(act.astype(jnp.bfloat16), dp_ref[0].astype(jnp.bfloat16),
                    preferred_element_type=jnp.float32)

    contrib = jnp.dot(selw, out_b, preferred_element_type=jnp.float32)

    @pl.when(e == 0)
    def _init():
        out_ref[...] = contrib

    @pl.when(e > 0)
    def _acc():
        out_ref[...] += contrib


def kernel(hidden_states, router_logits, gate_up_proj, down_proj):
    wmap = lambda s: (jnp.maximum(s, 1) - 1, 0, 0)
    return pl.pallas_call(
        _moe_kernel,
        grid=(E + 1,),
        in_specs=[
            pl.BlockSpec((T, D), lambda s: (0, 0)),
            pl.BlockSpec((T, E), lambda s: (0, 0)),
            pl.BlockSpec((1, D, 2 * FF), wmap),
            pl.BlockSpec((1, FF, D), wmap),
        ],
        out_specs=pl.BlockSpec((T, D), lambda e: (0, 0)),
        out_shape=jax.ShapeDtypeStruct((T, D), jnp.float32),
        scratch_shapes=[pltpu.VMEM((T, 8), jnp.float32)],
        compiler_params=pltpu.CompilerParams(
            dimension_semantics=("arbitrary",),
        ),
    )(hidden_states, router_logits, gate_up_proj, down_proj)
